# baseline (speedup 1.0000x reference)
import jax
import jax.numpy as jnp
from jax import lax
from jax.experimental import pallas as pl
from jax.experimental.pallas import tpu as pltpu
from jax.experimental.pallas import tpu_sc as plsc

SEQ_LEN = 4096
BATCH = 4
D_MODEL = 1024
LANES = 16

NC, NS = 2, 16
NW = NC * NS
ROWS = SEQ_LEN * BATCH
RPW = ROWS // NW
C = 16
NCHUNK = RPW // C
NJ = D_MODEL // LANES


def _body(x_hbm, idx_hbm, pe_hbm, out_hbm, idx_v, xbuf, pebuf, sem_x, sem_pe):
    wid = lax.axis_index("s") * NC + lax.axis_index("c")
    pltpu.sync_copy(idx_hbm.at[wid], idx_v)

    def chunk(c, carry):
        rowbase = wid * RPW + c * C
        cp_x = pltpu.async_copy(x_hbm.at[pl.ds(rowbase, C)], xbuf, sem_x)
        cp_pe = pltpu.async_copy(pe_hbm.at[idx_v.at[c]], pebuf, sem_pe)
        cp_x.wait()
        cp_pe.wait()

        def row(r, carry2):
            for j in range(NJ):
                sl = pl.ds(j * LANES, LANES)
                pebuf[r, sl] = pebuf[r, sl] + xbuf[r, sl]
            return carry2

        lax.fori_loop(0, C, row, 0)
        pltpu.sync_copy(pebuf, out_hbm.at[pl.ds(rowbase, C)])
        return carry

    lax.fori_loop(0, NCHUNK, chunk, 0)


@jax.jit
def _sc_add_gather(xf, idx3, pef):
    mesh = plsc.VectorSubcoreMesh(
        core_axis_name="c", subcore_axis_name="s",
        num_cores=NC, num_subcores=NS,
    )
    return pl.kernel(
        _body,
        out_type=jax.ShapeDtypeStruct((ROWS, D_MODEL), jnp.float32),
        mesh=mesh,
        scratch_types=[
            pltpu.VMEM((NCHUNK, C), jnp.int32),
            pltpu.VMEM((C, D_MODEL), jnp.float32),
            pltpu.VMEM((C, D_MODEL), jnp.float32),
            pltpu.SemaphoreType.DMA,
            pltpu.SemaphoreType.DMA,
        ],
    )(xf, idx3, pef)


def kernel(x, index, pe):
    xf = x.reshape(ROWS, D_MODEL)
    idx3 = index.astype(jnp.int32).T.reshape(NW, NCHUNK, C)
    pef = pe.reshape(SEQ_LEN, D_MODEL)
    out = _sc_add_gather(xf, idx3, pef)
    return out.reshape(SEQ_LEN, BATCH, D_MODEL)


# speedup vs baseline: 1.1629x; 1.1629x over previous
import jax
import jax.numpy as jnp
from jax import lax
from jax.experimental import pallas as pl
from jax.experimental.pallas import tpu as pltpu
from jax.experimental.pallas import tpu_sc as plsc

SEQ_LEN = 4096
BATCH = 4
D_MODEL = 1024
LANES = 16

NC, NS = 2, 16
NW = NC * NS
ROWS = SEQ_LEN * BATCH
RPW = ROWS // NW
C = 16
NCHUNK = RPW // C
NJ = D_MODEL // LANES
NBUF = 2
NG = NCHUNK // NBUF


def _body(x_hbm, idx_hbm, pe_hbm, out_hbm, idx_v, xbuf, pebuf, obuf,
          in_s0, in_s1, out_s0, out_s1):
    in_sems = (in_s0, in_s1)
    out_sems = (out_s0, out_s1)
    wid = lax.axis_index("s") * NC + lax.axis_index("c")
    rowbase0 = wid * RPW
    pltpu.sync_copy(idx_hbm.at[wid], idx_v)

    def start_in(b, c):
        rb = rowbase0 + c * C
        pltpu.async_copy(x_hbm.at[pl.ds(rb, C)], xbuf.at[b], in_sems[b])
        pltpu.async_copy(pe_hbm.at[idx_v.at[c]], pebuf.at[b], in_sems[b])

    def wait_in(b):
        pltpu.make_async_copy(x_hbm.at[pl.ds(0, C)], xbuf.at[b], in_sems[b]).wait()
        pltpu.make_async_copy(x_hbm.at[pl.ds(0, C)], pebuf.at[b], in_sems[b]).wait()

    def start_out(b, c):
        rb = rowbase0 + c * C
        pltpu.async_copy(obuf.at[b], out_hbm.at[pl.ds(rb, C)], out_sems[b])

    def wait_out(b):
        pltpu.make_async_copy(obuf.at[b], out_hbm.at[pl.ds(0, C)], out_sems[b]).wait()

    for b in range(NBUF):
        start_in(b, b)

    def group(g, carry):
        for b in range(NBUF):
            c = g * NBUF + b
            wait_in(b)
            pl.when(g >= 1)(lambda: wait_out(b))

            def row(r, carry2):
                for j in range(NJ):
                    sl = pl.ds(j * LANES, LANES)
                    obuf[b, r, sl] = xbuf[b, r, sl] + pebuf[b, r, sl]
                return carry2

            lax.fori_loop(0, C, row, 0)
            pl.when(g < NG - 1)(lambda: start_in(b, c + NBUF))
            start_out(b, c)
        return carry

    lax.fori_loop(0, NG, group, 0)
    for b in range(NBUF):
        wait_out(b)


@jax.jit
def _sc_add_gather(xf, idx3, pef):
    mesh = plsc.VectorSubcoreMesh(
        core_axis_name="c", subcore_axis_name="s",
        num_cores=NC, num_subcores=NS,
    )
    return pl.kernel(
        _body,
        out_type=jax.ShapeDtypeStruct((ROWS, D_MODEL), jnp.float32),
        mesh=mesh,
        scratch_types=[
            pltpu.VMEM((NCHUNK, C), jnp.int32),
            pltpu.VMEM((NBUF, C, D_MODEL), jnp.float32),
            pltpu.VMEM((NBUF, C, D_MODEL), jnp.float32),
            pltpu.VMEM((NBUF, C, D_MODEL), jnp.float32),
            pltpu.SemaphoreType.DMA,
            pltpu.SemaphoreType.DMA,
            pltpu.SemaphoreType.DMA,
            pltpu.SemaphoreType.DMA,
        ],
    )(xf, idx3, pef)


def kernel(x, index, pe):
    xf = x.reshape(ROWS, D_MODEL)
    idx3 = index.astype(jnp.int32).T.reshape(NW, NCHUNK, C)
    pef = pe.reshape(SEQ_LEN, D_MODEL)
    out = _sc_add_gather(xf, idx3, pef)
    return out.reshape(SEQ_LEN, BATCH, D_MODEL)


# speedup vs baseline: 3.3443x; 2.8757x over previous
import jax
import jax.numpy as jnp
from jax import lax
from jax.experimental import pallas as pl
from jax.experimental.pallas import tpu as pltpu
from jax.experimental.pallas import tpu_sc as plsc

SEQ_LEN = 4096
BATCH = 4
D_MODEL = 1024
LANES = 16

NC, NS = 2, 16
NW = NC * NS
LPW = SEQ_LEN // NW
CL = 4
NCHUNK = LPW // CL
CROWS = CL * 32
NBUF = 2
NG = NCHUNK // NBUF

XROWS = SEQ_LEN * 32
PEROWS = SEQ_LEN * 8

_GDN = lax.GatherDimensionNumbers(
    offset_dims=(), collapsed_slice_dims=(0,), start_index_map=(0,))


def _vgather(v, idx):
    return lax.gather(v, idx[:, None], dimension_numbers=_GDN,
                      slice_sizes=(1,),
                      mode=lax.GatherScatterMode.PROMISE_IN_BOUNDS)


def _body(x_hbm, idx_hbm, pe_hbm, out_hbm, idxq, lbuf, xbuf, pebuf, obuf,
          in_s0, in_s1, out_s0, out_s1):
    in_sems = (in_s0, in_s1)
    out_sems = (out_s0, out_s1)
    wid = lax.axis_index("s") * NC + lax.axis_index("c")
    xbase = wid * (LPW * 32)
    pltpu.sync_copy(idx_hbm.at[pl.ds(wid * BATCH, BATCH)], idxq)

    iota = lax.iota(jnp.int32, LANES)
    perm_lo = lax.bitwise_and(iota, 3)
    dt_hi = lax.shift_right_logical(iota, 2)

    def start_in(b, c):
        pltpu.async_copy(x_hbm.at[pl.ds(xbase + c * CROWS, CROWS)],
                         xbuf.at[b], in_sems[b])
        row = lax.shift_right_logical(c, 3)
        col = lax.bitwise_and(c, 7) * LANES
        raw = idxq[row, pl.ds(col, LANES)]
        for i in range(8):
            perm = perm_lo + 4 * (i // 2)
            dt = dt_hi + 4 * (i % 2)
            lbuf[b, pl.ds(i * LANES, LANES)] = _vgather(raw, perm) * 8 + dt
        pltpu.async_copy(pe_hbm.at[lbuf.at[b]], pebuf.at[b], in_sems[b])

    def wait_in(b):
        pltpu.make_async_copy(x_hbm.at[pl.ds(0, CROWS)], xbuf.at[b],
                              in_sems[b]).wait()
        pltpu.make_async_copy(x_hbm.at[pl.ds(0, CROWS)], pebuf.at[b],
                              in_sems[b]).wait()

    def start_out(b, c):
        pltpu.async_copy(obuf.at[b], out_hbm.at[pl.ds(xbase + c * CROWS, CROWS)],
                         out_sems[b])

    def wait_out(b):
        pltpu.make_async_copy(obuf.at[b], out_hbm.at[pl.ds(0, CROWS)],
                              out_sems[b]).wait()

    for b in range(NBUF):
        start_in(b, b)

    def group(g, carry):
        for b in range(NBUF):
            c = g * NBUF + b
            wait_in(b)
            pl.when(g >= 1)(lambda: wait_out(b))

            def rowadd(r, carry2):
                for v in range(8):
                    sl = pl.ds(v * LANES, LANES)
                    obuf[b, r, sl] = xbuf[b, r, sl] + pebuf[b, r, sl]
                return carry2

            lax.fori_loop(0, CROWS, rowadd, 0)
            pl.when(g < NG - 1)(lambda: start_in(b, c + NBUF))
            start_out(b, c)
        return carry

    lax.fori_loop(0, NG, group, 0)
    for b in range(NBUF):
        wait_out(b)


@jax.jit
def _sc_add_gather(xv, idxv, pev):
    mesh = plsc.VectorSubcoreMesh(
        core_axis_name="c", subcore_axis_name="s",
        num_cores=NC, num_subcores=NS,
    )
    return pl.kernel(
        _body,
        out_type=jax.ShapeDtypeStruct((XROWS, 128), jnp.float32),
        mesh=mesh,
        scratch_types=[
            pltpu.VMEM((BATCH, 128), jnp.int32),
            pltpu.VMEM((NBUF, CROWS), jnp.int32),
            pltpu.VMEM((NBUF, CROWS, 128), jnp.float32),
            pltpu.VMEM((NBUF, CROWS, 128), jnp.float32),
            pltpu.VMEM((NBUF, CROWS, 128), jnp.float32),
            pltpu.SemaphoreType.DMA,
            pltpu.SemaphoreType.DMA,
            pltpu.SemaphoreType.DMA,
            pltpu.SemaphoreType.DMA,
        ],
    )(xv, idxv, pev)


def kernel(x, index, pe):
    xv = jnp.transpose(x.reshape(SEQ_LEN, BATCH, 8, 128),
                       (0, 2, 1, 3)).reshape(XROWS, 128)
    pev = pe.reshape(PEROWS, 128)
    idxv = index.astype(jnp.int32).T.reshape(128, 128)
    o = _sc_add_gather(xv, idxv, pev)
    return jnp.transpose(o.reshape(SEQ_LEN, 8, BATCH, 128),
                         (0, 2, 1, 3)).reshape(SEQ_LEN, BATCH, D_MODEL)
